# baseline (speedup 1.0000x reference)
import jax
import jax.numpy as jnp
from jax.experimental import pallas as pl

_HIDDEN = 4096
_E = 64
_K = 8
_BT = 512


def _router_kernel(x_ref, w_ref, topw_ref, topi_ref, logits_ref):
    x = x_ref[...]
    w = w_ref[...]
    logits_t = jax.lax.dot_general(
        w, x,
        dimension_numbers=(((1,), (1,)), ((), ())),
        preferred_element_type=jnp.float32,
    )
    logits = logits_t.T
    logits_ref[...] = logits

    _RC = 64
    iota = jax.lax.broadcasted_iota(jnp.int32, (_RC, _E), 1)
    for c in range(_BT // _RC):
        l = logits[c * _RC:(c + 1) * _RC, :]
        m = jnp.max(l, axis=1, keepdims=True)
        e = jnp.exp(l - m)
        s = jnp.sum(e, axis=1, keepdims=True)
        vals = e / s
        tops = []
        idxs = []
        total = jnp.zeros((_RC, 1), jnp.float32)
        for _ in range(_K):
            mv = jnp.max(vals, axis=1, keepdims=True)
            ix = jnp.min(jnp.where(vals == mv, iota, _E), axis=1, keepdims=True)
            tops.append(mv)
            idxs.append(ix)
            total = total + mv
            vals = jnp.where(iota == ix, -jnp.inf, vals)
        for j in range(_K):
            topw_ref[c * _RC:(c + 1) * _RC, j:j + 1] = tops[j] / total
            topi_ref[c * _RC:(c + 1) * _RC, j:j + 1] = idxs[j]


@jax.jit
def kernel(x, W_gate):
    tokens = x.shape[0]
    grid = (tokens // _BT,)
    topw, topi, logits = pl.pallas_call(
        _router_kernel,
        grid=grid,
        in_specs=[
            pl.BlockSpec((_BT, _HIDDEN), lambda i: (i, 0)),
            pl.BlockSpec((_E, _HIDDEN), lambda i: (0, 0)),
        ],
        out_specs=[
            pl.BlockSpec((_BT, _K), lambda i: (i, 0)),
            pl.BlockSpec((_BT, _K), lambda i: (i, 0)),
            pl.BlockSpec((_BT, _E), lambda i: (i, 0)),
        ],
        out_shape=[
            jax.ShapeDtypeStruct((tokens, _K), jnp.float32),
            jax.ShapeDtypeStruct((tokens, _K), jnp.int32),
            jax.ShapeDtypeStruct((tokens, _E), jnp.float32),
        ],
    )(x, W_gate)
    return topw, topi, logits


# speedup vs baseline: 1.1722x; 1.1722x over previous
import jax
import jax.numpy as jnp
from jax.experimental import pallas as pl

_HIDDEN = 4096
_E = 64
_K = 8
_BT = 512


def _router_kernel(x_ref, w_ref, topw_ref, topi_ref, logits_ref):
    x = x_ref[...]
    w = w_ref[...]
    logits = jax.lax.dot_general(
        x, w,
        dimension_numbers=(((1,), (1,)), ((), ())),
        preferred_element_type=jnp.float32,
    )
    logits_ref[...] = logits

    _RC = 64
    iota = jax.lax.broadcasted_iota(jnp.int32, (_RC, _E), 1).astype(jnp.float32)
    for c in range(_BT // _RC):
        l = logits[c * _RC:(c + 1) * _RC, :]
        m = jnp.max(l, axis=1, keepdims=True)
        e = jnp.exp(l - m)
        s = jnp.sum(e, axis=1, keepdims=True)
        vals = e / s
        tops = []
        idxs = []
        total = jnp.zeros((_RC, 1), jnp.float32)
        for _ in range(_K):
            mv = jnp.max(vals, axis=1, keepdims=True)
            ix = jnp.min(jnp.where(vals == mv, iota, float(_E)),
                         axis=1, keepdims=True)
            tops.append(mv)
            idxs.append(ix)
            total = total + mv
            vals = jnp.where(iota == ix, -jnp.inf, vals)
        for j in range(_K):
            topw_ref[c * _RC:(c + 1) * _RC, j:j + 1] = tops[j] / total
            topi_ref[c * _RC:(c + 1) * _RC, j:j + 1] = idxs[j].astype(jnp.int32)


@jax.jit
def kernel(x, W_gate):
    tokens = x.shape[0]
    grid = (tokens // _BT,)
    topw, topi, logits = pl.pallas_call(
        _router_kernel,
        grid=grid,
        in_specs=[
            pl.BlockSpec((_BT, _HIDDEN), lambda i: (i, 0)),
            pl.BlockSpec((_E, _HIDDEN), lambda i: (0, 0)),
        ],
        out_specs=[
            pl.BlockSpec((_BT, _K), lambda i: (i, 0)),
            pl.BlockSpec((_BT, _K), lambda i: (i, 0)),
            pl.BlockSpec((_BT, _E), lambda i: (i, 0)),
        ],
        out_shape=[
            jax.ShapeDtypeStruct((tokens, _K), jnp.float32),
            jax.ShapeDtypeStruct((tokens, _K), jnp.int32),
            jax.ShapeDtypeStruct((tokens, _E), jnp.float32),
        ],
    )(x, W_gate)
    return topw, topi, logits
